# Initial kernel scaffold; baseline (speedup 1.0000x reference)
import jax
import jax.numpy as jnp
from jax.experimental import pallas as pl


def kernel(x, x_all, edge_index_1, edge_index_2, batch, batch_all, params):
    raise NotImplementedError("write your pallas kernel here")


# speedup vs baseline: 6.7012x; 6.7012x over previous
import jax
import jax.numpy as jnp
from jax import lax
from jax.experimental import pallas as pl
from jax.experimental.pallas import tpu as pltpu
from jax.experimental.pallas import tpu_sc as plsc

_N = 10000
_E = 320000
_D = 64
_G = 64
_FIN = 242
_FALL = 652

_NPAD = 10240
_DUMMY = 10200
_CH = 128
_NCH = 157
_EP = 16 * _NCH * _CH
_RP = 200
_NB = _N // (2 * _RP)


def _seg_body(ys, src_h, dst_h, agg, src_v, dst_v, rows0, rows1, acc,
              sem_i, sem0, sem1):
    c = lax.axis_index("c")
    s = lax.axis_index("s")
    cp_si = pltpu.async_copy(src_h.at[c, pl.ds(s * _NCH, _NCH)], src_v, sem_i)
    cp_di = pltpu.async_copy(dst_h.at[c, pl.ds(s * _NCH, _NCH)], dst_v, sem_i)

    @pl.loop(0, _CH)
    def _zero(r):
        for q in range(_D // 16):
            rows0[r, pl.ds(q * 16, 16)] = jnp.zeros((16,), jnp.float32)

    for k in range(_NPAD // 16 // _CH):
        pltpu.sync_copy(rows0, acc.at[pl.ds((s * 5 + k) * _CH, _CH)])
    cp_si.wait()
    cp_di.wait()
    plsc.subcore_barrier()

    @pl.loop(0, _NCH)
    def _edges(j):
        pltpu.async_copy(ys.at[src_v.at[j]], rows0, sem0).wait()
        pltpu.sync_copy(rows0, acc.at[dst_v.at[j]], add=True)

    plsc.subcore_barrier()
    rpt = _NPAD // 16
    pltpu.sync_copy(acc.at[pl.ds(s * rpt, rpt)],
                    agg.at[c, pl.ds(s * rpt, rpt)])


def _seg_sum(ys2, src_idx, dst_idx):
    mesh = plsc.VectorSubcoreMesh(core_axis_name="c", subcore_axis_name="s",
                                  num_cores=2, num_subcores=16)
    kern = pl.kernel(
        _seg_body,
        out_type=jax.ShapeDtypeStruct((2, _NPAD, _D), jnp.float32),
        mesh=mesh,
        scratch_types=[
            pltpu.VMEM((_NCH, _CH), jnp.int32),
            pltpu.VMEM((_NCH, _CH), jnp.int32),
            pltpu.VMEM((_CH, _D), jnp.float32),
            pltpu.VMEM((_CH, _D), jnp.float32),
            pltpu.VMEM_SHARED((_NPAD, _D), jnp.float32),
            pltpu.SemaphoreType.DMA,
            pltpu.SemaphoreType.DMA,
            pltpu.SemaphoreType.DMA,
        ],
        compiler_params=pltpu.CompilerParams(use_tc_tiling_on_sc=False),
    )
    return kern(ys2, src_idx, dst_idx)


def _lin2_body(h_ref, ab_ref, w_ref, o_ref):
    k = h_ref.shape[1] // 2
    a = ab_ref[0:1, :]
    b = ab_ref[1:2, :]
    he = h_ref[:, :k] * a + b
    ho = h_ref[:, k:] * a + b
    w = w_ref[...]
    o_ref[0] = jnp.dot(he, w, preferred_element_type=jnp.float32)
    o_ref[1] = jnp.dot(ho, w, preferred_element_type=jnp.float32)


def _lin2(h, ab, w):
    k = w.shape[0]
    return pl.pallas_call(
        _lin2_body,
        grid=(_NB,),
        in_specs=[
            pl.BlockSpec((_RP, 2 * k), lambda i: (i, 0)),
            pl.BlockSpec((2, k), lambda i: (0, 0)),
            pl.BlockSpec((k, 2 * _D), lambda i: (0, 0)),
        ],
        out_specs=pl.BlockSpec((2, _RP, 2 * _D), lambda i: (0, i, 0)),
        out_shape=jax.ShapeDtypeStruct((2, _N // 2, 2 * _D), jnp.float32),
    )(h, ab, w)


def _post_body(ys_ref, s1_ref, s2_ref, eb_ref, b1_ref, w2_ref, b2_ref,
               wm1_ref, bm1_ref, wm2_ref, bm2_ref, gb_ref,
               r_ref, ab_ref, stat_ref):
    i = pl.program_id(0)
    scat_e = jnp.concatenate([s1_ref[0][:, :_D], s2_ref[0][:, :_D]], axis=1)
    scat_o = jnp.concatenate([s1_ref[0][:, _D:], s2_ref[0][:, _D:]], axis=1)
    eb = eb_ref[...]
    b1 = b1_ref[...]
    w2 = w2_ref[...]
    b2 = b2_ref[...]
    wm1 = wm1_ref[...]
    bm1 = bm1_ref[...]
    wm2 = wm2_ref[...]
    bm2 = bm2_ref[...]

    def half(ys, scat):
        z = jnp.maximum(ys * eb + scat + b1, 0.0)
        u = jnp.maximum(
            jnp.dot(z, w2, preferred_element_type=jnp.float32) + b2, 0.0)
        v = jnp.maximum(
            jnp.dot(u, wm1, preferred_element_type=jnp.float32) + bm1, 0.0)
        return jnp.dot(v, wm2, preferred_element_type=jnp.float32) + bm2

    re = half(ys_ref[0], scat_e)
    ro = half(ys_ref[1], scat_o)
    r_ref[...] = jnp.concatenate([re, ro], axis=1)

    @pl.when(i == 0)
    def _():
        stat_ref[...] = jnp.zeros_like(stat_ref)

    stat_ref[0:1, :] += (jnp.sum(re, axis=0, keepdims=True)
                         + jnp.sum(ro, axis=0, keepdims=True))
    stat_ref[1:2, :] += (jnp.sum(re * re, axis=0, keepdims=True)
                         + jnp.sum(ro * ro, axis=0, keepdims=True))

    @pl.when(i == _NB - 1)
    def _():
        mu = stat_ref[0:1, :] * (1.0 / _N)
        var = stat_ref[1:2, :] * (1.0 / _N) - mu * mu
        a = gb_ref[0:1, :] * lax.rsqrt(var + 1e-5)
        ab_ref[0:1, :] = a
        ab_ref[1:2, :] = gb_ref[1:2, :] - mu * a


def _post(ys, agg, eb, b1r, w2bd, b2r, wm1, bm1r, wm2, bm2r, gb):
    return pl.pallas_call(
        _post_body,
        grid=(_NB,),
        in_specs=[
            pl.BlockSpec((2, _RP, 2 * _D), lambda i: (0, i, 0)),
            pl.BlockSpec((1, _RP, 2 * _D), lambda i: (0, i, 0)),
            pl.BlockSpec((1, _RP, 2 * _D), lambda i: (1, i, 0)),
            pl.BlockSpec((1, 2 * _D), lambda i: (0, 0)),
            pl.BlockSpec((1, 2 * _D), lambda i: (0, 0)),
            pl.BlockSpec((2 * _D, 2 * _D), lambda i: (0, 0)),
            pl.BlockSpec((1, 2 * _D), lambda i: (0, 0)),
            pl.BlockSpec((2 * _D, _D), lambda i: (0, 0)),
            pl.BlockSpec((1, _D), lambda i: (0, 0)),
            pl.BlockSpec((_D, _D), lambda i: (0, 0)),
            pl.BlockSpec((1, _D), lambda i: (0, 0)),
            pl.BlockSpec((2, _D), lambda i: (0, 0)),
        ],
        out_specs=[
            pl.BlockSpec((_RP, 2 * _D), lambda i: (i, 0)),
            pl.BlockSpec((2, _D), lambda i: (0, 0)),
        ],
        out_shape=[
            jax.ShapeDtypeStruct((_N // 2, 2 * _D), jnp.float32),
            jax.ShapeDtypeStruct((2, _D), jnp.float32),
        ],
        scratch_shapes=[pltpu.VMEM((2, _D), jnp.float32)],
    )(ys, agg, agg, eb, b1r, w2bd, b2r, wm1, bm1r, wm2, bm2r, gb)


def _pool_body(b2_ref, bp_ref, r1_ref, r2_ref, r3_ref, r4_ref, xa_ref,
               p_ref, pa_ref, cnt_ref):
    i = pl.program_id(0)

    @pl.when(i == 0)
    def _():
        p_ref[...] = jnp.zeros_like(p_ref)
        pa_ref[...] = jnp.zeros_like(pa_ref)
        cnt_ref[...] = jnp.zeros_like(cnt_ref)

    dn = (((0,), (0,)), ((), ()))
    iota2 = lax.broadcasted_iota(jnp.int32, (2 * _RP, _G), 1)
    iota1 = lax.broadcasted_iota(jnp.int32, (_RP, _G), 1)
    oh = (iota2 == b2_ref[:, 0:1]).astype(jnp.float32)
    oha = (iota2 == b2_ref[:, 1:2]).astype(jnp.float32)
    ohe = (iota1 == bp_ref[:, 0:1]).astype(jnp.float32)
    oho = (iota1 == bp_ref[:, 2:3]).astype(jnp.float32)
    for l, rref in enumerate((r1_ref, r2_ref, r3_ref, r4_ref)):
        rp = rref[...]
        p_ref[:, l * _D:(l + 1) * _D] += (
            lax.dot_general(ohe, rp[:, :_D], dn,
                            preferred_element_type=jnp.float32)
            + lax.dot_general(oho, rp[:, _D:], dn,
                              preferred_element_type=jnp.float32))
    pa_ref[...] += lax.dot_general(
        oha, xa_ref[...], dn, preferred_element_type=jnp.float32)
    ones8 = jnp.ones((2 * _RP, 8), jnp.float32)
    cnt_ref[:, 0:8] += lax.dot_general(
        oh, ones8, dn, preferred_element_type=jnp.float32)
    cnt_ref[:, 8:16] += lax.dot_general(
        oha, ones8, dn, preferred_element_type=jnp.float32)


def _pool(batch2, bp, r1, r2, r3, r4, xap):
    fa = xap.shape[1]
    rspec = pl.BlockSpec((_RP, 2 * _D), lambda i: (i, 0))
    return pl.pallas_call(
        _pool_body,
        grid=(_NB,),
        in_specs=[
            pl.BlockSpec((2 * _RP, 2), lambda i: (i, 0)),
            pl.BlockSpec((_RP, 4), lambda i: (i, 0)),
            rspec, rspec, rspec, rspec,
            pl.BlockSpec((2 * _RP, fa), lambda i: (i, 0)),
        ],
        out_specs=[
            pl.BlockSpec((_G, 4 * _D), lambda i: (0, 0)),
            pl.BlockSpec((_G, fa), lambda i: (0, 0)),
            pl.BlockSpec((_G, 16), lambda i: (0, 0)),
        ],
        out_shape=[
            jax.ShapeDtypeStruct((_G, 4 * _D), jnp.float32),
            jax.ShapeDtypeStruct((_G, fa), jnp.float32),
            jax.ShapeDtypeStruct((_G, 16), jnp.float32),
        ],
    )(batch2, bp, r1, r2, r3, r4, xap)


def _final_body(p_ref, pa_ref, cnt_ref, a_ref, b_ref, w1a_ref, w1b_ref,
                fb1_ref, w2_ref, fb2_ref, w3_ref, fb3_ref, w4_ref, fb4_ref,
                o_ref):
    ca = cnt_ref[:, 0:1]
    cb = cnt_ref[:, 8:9]
    mask = (ca > 0.0).astype(jnp.float32)
    avg = p_ref[...] / jnp.maximum(ca, 1.0)
    xg = (avg * a_ref[...] + b_ref[...]) * mask
    xag = pa_ref[...] / jnp.maximum(cb, 1.0)
    z = jnp.maximum(
        jnp.dot(xg, w1a_ref[...], preferred_element_type=jnp.float32)
        + jnp.dot(xag, w1b_ref[...], preferred_element_type=jnp.float32)
        + fb1_ref[...], 0.0)
    z = jnp.maximum(
        jnp.dot(z, w2_ref[...], preferred_element_type=jnp.float32)
        + fb2_ref[...], 0.0)
    z = jnp.maximum(
        jnp.dot(z, w3_ref[...], preferred_element_type=jnp.float32)
        + fb3_ref[...], 0.0)
    o_ref[...] = (jnp.dot(z, w4_ref[...], preferred_element_type=jnp.float32)
                  + fb4_ref[...])


def _final(p, pa, cnt, arow, brow, w1a, w1b, fb1, w2, fb2, w3, fb3, w4, fb4):
    fa = pa.shape[1]
    full = lambda shape: pl.BlockSpec(shape, lambda: tuple(0 for _ in shape))
    return pl.pallas_call(
        _final_body,
        grid=(),
        in_specs=[
            full((_G, 4 * _D)), full((_G, fa)), full((_G, 16)),
            full((1, 4 * _D)), full((1, 4 * _D)),
            full((4 * _D, _D)), full((fa, _D)), full((1, _D)),
            full((_D, _D)), full((1, _D)),
            full((_D, _D)), full((1, _D)),
            full((_D, 2 * _D)), full((1, 2 * _D)),
        ],
        out_specs=full((_G, 2 * _D)),
        out_shape=jax.ShapeDtypeStruct((_G, 2 * _D), jnp.float32),
    )(p, pa, cnt, arow, brow, w1a, w1b, fb1, w2, fb2, w3, fb3, w4, fb4)


def kernel(x, x_all, edge_index_1, edge_index_2, batch, batch_all, params):
    f32 = jnp.float32
    p = params

    def prep(ei, off):
        n = ei[0]
        src = n + off + (n % 2) * (_N - 1)
        dst = ei[1]
        src = jnp.concatenate([src, jnp.zeros((_EP - _E,), jnp.int32)])
        dst = jnp.concatenate([dst, jnp.full((_EP - _E,), _DUMMY, jnp.int32)])
        return src.reshape(16 * _NCH, _CH), dst.reshape(16 * _NCH, _CH)

    s1, d1 = prep(edge_index_1, 0)
    s2, d2 = prep(edge_index_2, 1)
    src_idx = jnp.stack([s1, s2])
    dst_idx = jnp.stack([d1, d2])

    xp = jnp.pad(x, ((0, 0), (0, 256 - _FIN))).reshape(_N // 2, 512)
    xap = jnp.pad(x_all, ((0, 0), (0, 768 - _FALL)))
    batch2 = jnp.stack([batch, batch_all], axis=1)
    bp = batch2.reshape(_N // 2, 4)

    h = xp
    ab = jnp.concatenate(
        [jnp.ones((1, 256), f32), jnp.zeros((1, 256), f32)], axis=0)
    a_parts, b_parts, rs = [], [], []
    for L in range(1, 5):
        c1 = p["conv%d_1" % L]
        c2 = p["conv%d_2" % L]
        m = p["mlp%d" % L]
        bn = p["bn%d" % L]
        w1 = jnp.concatenate([c1["W1"], c2["W1"]], axis=1)
        if L == 1:
            w1 = jnp.pad(w1, ((0, 256 - _FIN), (0, 0)))
        ysp = _lin2(h, ab, w1)
        agg = _seg_sum(ysp.reshape(2 * _N, _D), src_idx, dst_idx)
        agg = agg.reshape(2, _NPAD // 2, 2 * _D)
        eb = jnp.concatenate([jnp.full((1, _D), 1.0 + c1["eps"], f32),
                              jnp.full((1, _D), 1.0 + c2["eps"], f32)], axis=1)
        b1r = jnp.concatenate([c1["b1"], c2["b1"]]).reshape(1, 2 * _D)
        w2bd = (jnp.zeros((2 * _D, 2 * _D), f32)
                .at[:_D, :_D].set(c1["W2"]).at[_D:, _D:].set(c2["W2"]))
        b2r = jnp.concatenate([c1["b2"], c2["b2"]]).reshape(1, 2 * _D)
        gb = jnp.stack([bn["gamma"], bn["beta"]])
        r, ab = _post(ysp, agg, eb, b1r, w2bd, b2r,
                      m["W1"], m["b1"].reshape(1, _D),
                      m["W2"], m["b2"].reshape(1, _D), gb)
        a_parts.append(ab[0:1, :])
        b_parts.append(ab[1:2, :])
        rs.append(r)
        h = r

    pg, pa, cnt = _pool(batch2, bp, rs[0], rs[1], rs[2], rs[3], xap)
    arow = jnp.concatenate(a_parts, axis=1)
    brow = jnp.concatenate(b_parts, axis=1)
    w1a = p["fc1"]["W"][:4 * _D]
    w1b = jnp.pad(p["fc1"]["W"][4 * _D:], ((0, 768 - _FALL), (0, 0)))
    fb1 = p["fc1"]["b"].reshape(1, _D)
    w4 = jnp.pad(p["fc4"]["W"], ((0, 0), (0, 2 * _D - 1)))
    fb4 = jnp.pad(p["fc4"]["b"], (0, 2 * _D - 1)).reshape(1, 2 * _D)
    out = _final(pg, pa, cnt, arow, brow, w1a, w1b, fb1,
                 p["fc2"]["W"], p["fc2"]["b"].reshape(1, _D),
                 p["fc3"]["W"], p["fc3"]["b"].reshape(1, _D), w4, fb4)
    return out[:, 0]


# speedup vs baseline: 6.7076x; 1.0010x over previous
import jax
import jax.numpy as jnp
from jax import lax
from jax.experimental import pallas as pl
from jax.experimental.pallas import tpu as pltpu
from jax.experimental.pallas import tpu_sc as plsc

_N = 10000
_E = 320000
_D = 64
_G = 64
_FIN = 242
_FALL = 652

_NPAD = 10240
_DUMMY = 10200
_CH = 128
_NCH = 157
_EP = 16 * _NCH * _CH
_RP = 200
_NB = _N // (2 * _RP)


def _seg_body(ys, src_h, dst_h, agg, src_v, dst_v, rows0, rows1, acc,
              sem_i, sem0, sem1):
    c = lax.axis_index("c")
    s = lax.axis_index("s")
    cp_si = pltpu.async_copy(src_h.at[c, s], src_v, sem_i)
    cp_di = pltpu.async_copy(dst_h.at[c, s], dst_v, sem_i)

    @pl.loop(0, _CH)
    def _zero(r):
        for q in range(_D // 16):
            rows0[r, pl.ds(q * 16, 16)] = jnp.zeros((16,), jnp.float32)

    for k in range(_NPAD // 16 // _CH):
        pltpu.sync_copy(rows0, acc.at[pl.ds((s * 5 + k) * _CH, _CH)])
    cp_si.wait()
    cp_di.wait()
    plsc.subcore_barrier()

    @pl.loop(0, _NCH)
    def _edges(j):
        pltpu.async_copy(ys.at[src_v.at[j]], rows0, sem0).wait()
        pltpu.sync_copy(rows0, acc.at[dst_v.at[j]], add=True)

    plsc.subcore_barrier()
    rpt = _NPAD // 16
    pltpu.sync_copy(acc.at[pl.ds(s * rpt, rpt)],
                    agg.at[c, pl.ds(s * rpt, rpt)])


def _seg_sum(ys2, src_idx, dst_idx):
    mesh = plsc.VectorSubcoreMesh(core_axis_name="c", subcore_axis_name="s",
                                  num_cores=2, num_subcores=16)
    kern = pl.kernel(
        _seg_body,
        out_type=jax.ShapeDtypeStruct((2, _NPAD, _D), jnp.float32),
        mesh=mesh,
        scratch_types=[
            pltpu.VMEM((_NCH, _CH), jnp.int32),
            pltpu.VMEM((_NCH, _CH), jnp.int32),
            pltpu.VMEM((_CH, _D), jnp.float32),
            pltpu.VMEM((_CH, _D), jnp.float32),
            pltpu.VMEM_SHARED((_NPAD, _D), jnp.float32),
            pltpu.SemaphoreType.DMA,
            pltpu.SemaphoreType.DMA,
            pltpu.SemaphoreType.DMA,
        ],
        compiler_params=pltpu.CompilerParams(use_tc_tiling_on_sc=False),
    )
    return kern(ys2, src_idx, dst_idx)


def _lin2_body(h_ref, ab_ref, w_ref, o_ref):
    k = h_ref.shape[1] // 2
    a = ab_ref[0:1, :]
    b = ab_ref[1:2, :]
    he = h_ref[:, :k] * a + b
    ho = h_ref[:, k:] * a + b
    w = w_ref[...]
    o_ref[0] = jnp.dot(he, w, preferred_element_type=jnp.float32)
    o_ref[1] = jnp.dot(ho, w, preferred_element_type=jnp.float32)


def _lin2(h, ab, w):
    k = w.shape[0]
    return pl.pallas_call(
        _lin2_body,
        grid=(_NB,),
        in_specs=[
            pl.BlockSpec((_RP, 2 * k), lambda i: (i, 0)),
            pl.BlockSpec((2, k), lambda i: (0, 0)),
            pl.BlockSpec((k, 2 * _D), lambda i: (0, 0)),
        ],
        out_specs=pl.BlockSpec((2, _RP, 2 * _D), lambda i: (0, i, 0)),
        out_shape=jax.ShapeDtypeStruct((2, _N // 2, 2 * _D), jnp.float32),
    )(h, ab, w)


def _post_body(ys_ref, s1_ref, s2_ref, eb_ref, b1_ref, w2_ref, b2_ref,
               wm1_ref, bm1_ref, wm2_ref, bm2_ref, gb_ref,
               r_ref, ab_ref, stat_ref):
    i = pl.program_id(0)
    scat_e = jnp.concatenate([s1_ref[0][:, :_D], s2_ref[0][:, :_D]], axis=1)
    scat_o = jnp.concatenate([s1_ref[0][:, _D:], s2_ref[0][:, _D:]], axis=1)
    eb = eb_ref[...]
    b1 = b1_ref[...]
    w2 = w2_ref[...]
    b2 = b2_ref[...]
    wm1 = wm1_ref[...]
    bm1 = bm1_ref[...]
    wm2 = wm2_ref[...]
    bm2 = bm2_ref[...]

    def half(ys, scat):
        z = jnp.maximum(ys * eb + scat + b1, 0.0)
        u = jnp.maximum(
            jnp.dot(z, w2, preferred_element_type=jnp.float32) + b2, 0.0)
        v = jnp.maximum(
            jnp.dot(u, wm1, preferred_element_type=jnp.float32) + bm1, 0.0)
        return jnp.dot(v, wm2, preferred_element_type=jnp.float32) + bm2

    re = half(ys_ref[0], scat_e)
    ro = half(ys_ref[1], scat_o)
    r_ref[...] = jnp.concatenate([re, ro], axis=1)

    @pl.when(i == 0)
    def _():
        stat_ref[...] = jnp.zeros_like(stat_ref)

    stat_ref[0:1, :] += (jnp.sum(re, axis=0, keepdims=True)
                         + jnp.sum(ro, axis=0, keepdims=True))
    stat_ref[1:2, :] += (jnp.sum(re * re, axis=0, keepdims=True)
                         + jnp.sum(ro * ro, axis=0, keepdims=True))

    @pl.when(i == _NB - 1)
    def _():
        mu = stat_ref[0:1, :] * (1.0 / _N)
        var = stat_ref[1:2, :] * (1.0 / _N) - mu * mu
        a = gb_ref[0:1, :] * lax.rsqrt(var + 1e-5)
        ab_ref[0:1, :] = a
        ab_ref[1:2, :] = gb_ref[1:2, :] - mu * a


def _post(ys, agg, eb, b1r, w2bd, b2r, wm1, bm1r, wm2, bm2r, gb):
    return pl.pallas_call(
        _post_body,
        grid=(_NB,),
        in_specs=[
            pl.BlockSpec((2, _RP, 2 * _D), lambda i: (0, i, 0)),
            pl.BlockSpec((1, _RP, 2 * _D), lambda i: (0, i, 0)),
            pl.BlockSpec((1, _RP, 2 * _D), lambda i: (1, i, 0)),
            pl.BlockSpec((1, 2 * _D), lambda i: (0, 0)),
            pl.BlockSpec((1, 2 * _D), lambda i: (0, 0)),
            pl.BlockSpec((2 * _D, 2 * _D), lambda i: (0, 0)),
            pl.BlockSpec((1, 2 * _D), lambda i: (0, 0)),
            pl.BlockSpec((2 * _D, _D), lambda i: (0, 0)),
            pl.BlockSpec((1, _D), lambda i: (0, 0)),
            pl.BlockSpec((_D, _D), lambda i: (0, 0)),
            pl.BlockSpec((1, _D), lambda i: (0, 0)),
            pl.BlockSpec((2, _D), lambda i: (0, 0)),
        ],
        out_specs=[
            pl.BlockSpec((_RP, 2 * _D), lambda i: (i, 0)),
            pl.BlockSpec((2, _D), lambda i: (0, 0)),
        ],
        out_shape=[
            jax.ShapeDtypeStruct((_N // 2, 2 * _D), jnp.float32),
            jax.ShapeDtypeStruct((2, _D), jnp.float32),
        ],
        scratch_shapes=[pltpu.VMEM((2, _D), jnp.float32)],
    )(ys, agg, agg, eb, b1r, w2bd, b2r, wm1, bm1r, wm2, bm2r, gb)


def _pool_body(b2_ref, bp_ref, r1_ref, r2_ref, r3_ref, r4_ref, xa_ref,
               p_ref, pa_ref, cnt_ref):
    i = pl.program_id(0)

    @pl.when(i == 0)
    def _():
        p_ref[...] = jnp.zeros_like(p_ref)
        pa_ref[...] = jnp.zeros_like(pa_ref)
        cnt_ref[...] = jnp.zeros_like(cnt_ref)

    dn = (((0,), (0,)), ((), ()))
    iota2 = lax.broadcasted_iota(jnp.int32, (2 * _RP, _G), 1)
    iota1 = lax.broadcasted_iota(jnp.int32, (_RP, _G), 1)
    oh = (iota2 == b2_ref[:, 0:1]).astype(jnp.float32)
    oha = (iota2 == b2_ref[:, 1:2]).astype(jnp.float32)
    ohe = (iota1 == bp_ref[:, 0:1]).astype(jnp.float32)
    oho = (iota1 == bp_ref[:, 2:3]).astype(jnp.float32)
    for l, rref in enumerate((r1_ref, r2_ref, r3_ref, r4_ref)):
        rp = rref[...]
        p_ref[:, l * _D:(l + 1) * _D] += (
            lax.dot_general(ohe, rp[:, :_D], dn,
                            preferred_element_type=jnp.float32)
            + lax.dot_general(oho, rp[:, _D:], dn,
                              preferred_element_type=jnp.float32))
    pa_ref[...] += lax.dot_general(
        oha, xa_ref[...], dn, preferred_element_type=jnp.float32)
    ones8 = jnp.ones((2 * _RP, 8), jnp.float32)
    cnt_ref[:, 0:8] += lax.dot_general(
        oh, ones8, dn, preferred_element_type=jnp.float32)
    cnt_ref[:, 8:16] += lax.dot_general(
        oha, ones8, dn, preferred_element_type=jnp.float32)


def _pool(batch2, bp, r1, r2, r3, r4, xap):
    fa = xap.shape[1]
    rspec = pl.BlockSpec((_RP, 2 * _D), lambda i: (i, 0))
    return pl.pallas_call(
        _pool_body,
        grid=(_NB,),
        in_specs=[
            pl.BlockSpec((2 * _RP, 2), lambda i: (i, 0)),
            pl.BlockSpec((_RP, 4), lambda i: (i, 0)),
            rspec, rspec, rspec, rspec,
            pl.BlockSpec((2 * _RP, fa), lambda i: (i, 0)),
        ],
        out_specs=[
            pl.BlockSpec((_G, 4 * _D), lambda i: (0, 0)),
            pl.BlockSpec((_G, fa), lambda i: (0, 0)),
            pl.BlockSpec((_G, 16), lambda i: (0, 0)),
        ],
        out_shape=[
            jax.ShapeDtypeStruct((_G, 4 * _D), jnp.float32),
            jax.ShapeDtypeStruct((_G, fa), jnp.float32),
            jax.ShapeDtypeStruct((_G, 16), jnp.float32),
        ],
    )(batch2, bp, r1, r2, r3, r4, xap)


def _final_body(p_ref, pa_ref, cnt_ref, a_ref, b_ref, w1a_ref, w1b_ref,
                fb1_ref, w2_ref, fb2_ref, w3_ref, fb3_ref, w4_ref, fb4_ref,
                o_ref):
    ca = cnt_ref[:, 0:1]
    cb = cnt_ref[:, 8:9]
    mask = (ca > 0.0).astype(jnp.float32)
    avg = p_ref[...] / jnp.maximum(ca, 1.0)
    xg = (avg * a_ref[...] + b_ref[...]) * mask
    xag = pa_ref[...] / jnp.maximum(cb, 1.0)
    z = jnp.maximum(
        jnp.dot(xg, w1a_ref[...], preferred_element_type=jnp.float32)
        + jnp.dot(xag, w1b_ref[...], preferred_element_type=jnp.float32)
        + fb1_ref[...], 0.0)
    z = jnp.maximum(
        jnp.dot(z, w2_ref[...], preferred_element_type=jnp.float32)
        + fb2_ref[...], 0.0)
    z = jnp.maximum(
        jnp.dot(z, w3_ref[...], preferred_element_type=jnp.float32)
        + fb3_ref[...], 0.0)
    o_ref[...] = (jnp.dot(z, w4_ref[...], preferred_element_type=jnp.float32)
                  + fb4_ref[...])


def _final(p, pa, cnt, arow, brow, w1a, w1b, fb1, w2, fb2, w3, fb3, w4, fb4):
    fa = pa.shape[1]
    full = lambda shape: pl.BlockSpec(shape, lambda: tuple(0 for _ in shape))
    return pl.pallas_call(
        _final_body,
        grid=(),
        in_specs=[
            full((_G, 4 * _D)), full((_G, fa)), full((_G, 16)),
            full((1, 4 * _D)), full((1, 4 * _D)),
            full((4 * _D, _D)), full((fa, _D)), full((1, _D)),
            full((_D, _D)), full((1, _D)),
            full((_D, _D)), full((1, _D)),
            full((_D, 2 * _D)), full((1, 2 * _D)),
        ],
        out_specs=full((_G, 2 * _D)),
        out_shape=jax.ShapeDtypeStruct((_G, 2 * _D), jnp.float32),
    )(p, pa, cnt, arow, brow, w1a, w1b, fb1, w2, fb2, w3, fb3, w4, fb4)


def kernel(x, x_all, edge_index_1, edge_index_2, batch, batch_all, params):
    f32 = jnp.float32
    p = params

    def prep(ei, off):
        n = ei[0]
        src = n + off + (n % 2) * (_N - 1)
        dst = ei[1]
        src = jnp.concatenate([src, jnp.zeros((_EP - _E,), jnp.int32)])
        dst = jnp.concatenate([dst, jnp.full((_EP - _E,), _DUMMY, jnp.int32)])
        return src.reshape(16, _NCH, _CH), dst.reshape(16, _NCH, _CH)

    s1, d1 = prep(edge_index_1, 0)
    s2, d2 = prep(edge_index_2, 1)
    src_idx = jnp.stack([s1, s2])
    dst_idx = jnp.stack([d1, d2])

    xp = jnp.pad(x, ((0, 0), (0, 256 - _FIN))).reshape(_N // 2, 512)
    xap = jnp.pad(x_all, ((0, 0), (0, 768 - _FALL)))
    batch2 = jnp.stack([batch, batch_all], axis=1)
    bp = batch2.reshape(_N // 2, 4)

    h = xp
    ab = jnp.concatenate(
        [jnp.ones((1, 256), f32), jnp.zeros((1, 256), f32)], axis=0)
    a_parts, b_parts, rs = [], [], []
    for L in range(1, 5):
        c1 = p["conv%d_1" % L]
        c2 = p["conv%d_2" % L]
        m = p["mlp%d" % L]
        bn = p["bn%d" % L]
        w1 = jnp.concatenate([c1["W1"], c2["W1"]], axis=1)
        if L == 1:
            w1 = jnp.pad(w1, ((0, 256 - _FIN), (0, 0)))
        ysp = _lin2(h, ab, w1)
        agg = _seg_sum(ysp.reshape(2 * _N, _D), src_idx, dst_idx)
        agg = agg.reshape(2, _NPAD // 2, 2 * _D)
        eb = jnp.concatenate([jnp.full((1, _D), 1.0 + c1["eps"], f32),
                              jnp.full((1, _D), 1.0 + c2["eps"], f32)], axis=1)
        b1r = jnp.concatenate([c1["b1"], c2["b1"]]).reshape(1, 2 * _D)
        w2bd = (jnp.zeros((2 * _D, 2 * _D), f32)
                .at[:_D, :_D].set(c1["W2"]).at[_D:, _D:].set(c2["W2"]))
        b2r = jnp.concatenate([c1["b2"], c2["b2"]]).reshape(1, 2 * _D)
        gb = jnp.stack([bn["gamma"], bn["beta"]])
        r, ab = _post(ysp, agg, eb, b1r, w2bd, b2r,
                      m["W1"], m["b1"].reshape(1, _D),
                      m["W2"], m["b2"].reshape(1, _D), gb)
        a_parts.append(ab[0:1, :])
        b_parts.append(ab[1:2, :])
        rs.append(r)
        h = r

    pg, pa, cnt = _pool(batch2, bp, rs[0], rs[1], rs[2], rs[3], xap)
    arow = jnp.concatenate(a_parts, axis=1)
    brow = jnp.concatenate(b_parts, axis=1)
    w1a = p["fc1"]["W"][:4 * _D]
    w1b = jnp.pad(p["fc1"]["W"][4 * _D:], ((0, 768 - _FALL), (0, 0)))
    fb1 = p["fc1"]["b"].reshape(1, _D)
    w4 = jnp.pad(p["fc4"]["W"], ((0, 0), (0, 2 * _D - 1)))
    fb4 = jnp.pad(p["fc4"]["b"], (0, 2 * _D - 1)).reshape(1, 2 * _D)
    out = _final(pg, pa, cnt, arow, brow, w1a, w1b, fb1,
                 p["fc2"]["W"], p["fc2"]["b"].reshape(1, _D),
                 p["fc3"]["W"], p["fc3"]["b"].reshape(1, _D), w4, fb4)
    return out[:, 0]


# speedup vs baseline: 6.7092x; 1.0002x over previous
import jax
import jax.numpy as jnp
from jax import lax
from jax.experimental import pallas as pl
from jax.experimental.pallas import tpu as pltpu
from jax.experimental.pallas import tpu_sc as plsc

_N = 10000
_E = 320000
_D = 64
_G = 64
_FIN = 242
_FALL = 652

_NPAD = 10240
_DUMMY = 10200
_CH = 128
_NCH = 157
_EP = 16 * _NCH * _CH
_RP = 200
_NB = _N // (2 * _RP)


def _seg_body(ys, src_h, dst_h, agg, src_v, dst_v, rows0, rows1, acc,
              sem_i, sem0, sem1):
    c = lax.axis_index("c")
    s = lax.axis_index("s")
    cp_si = pltpu.async_copy(src_h.at[c, s], src_v, sem_i)
    cp_di = pltpu.async_copy(dst_h.at[c, s], dst_v, sem_i)

    @pl.loop(0, _CH)
    def _zero(r):
        for q in range(_D // 16):
            rows0[r, pl.ds(q * 16, 16)] = jnp.zeros((16,), jnp.float32)

    for k in range(_NPAD // 16 // _CH):
        pltpu.sync_copy(rows0, acc.at[pl.ds((s * 5 + k) * _CH, _CH)])
    cp_si.wait()
    cp_di.wait()
    plsc.subcore_barrier()

    @pl.loop(0, _NCH)
    def _edges(j):
        pltpu.async_copy(ys.at[src_v.at[j]], rows0, sem0).wait()
        pltpu.sync_copy(rows0, acc.at[dst_v.at[j]], add=True)

    plsc.subcore_barrier()
    rpt = _NPAD // 16
    pltpu.sync_copy(acc.at[pl.ds(s * rpt, rpt)],
                    agg.at[c, pl.ds(s * rpt, rpt)])


def _seg_sum(ys2, src_idx, dst_idx):
    mesh = plsc.VectorSubcoreMesh(core_axis_name="c", subcore_axis_name="s",
                                  num_cores=2, num_subcores=16)
    kern = pl.kernel(
        _seg_body,
        out_type=jax.ShapeDtypeStruct((2, _NPAD, _D), jnp.float32),
        mesh=mesh,
        scratch_types=[
            pltpu.VMEM((_NCH, _CH), jnp.int32),
            pltpu.VMEM((_NCH, _CH), jnp.int32),
            pltpu.VMEM((_CH, _D), jnp.float32),
            pltpu.VMEM((_CH, _D), jnp.float32),
            pltpu.VMEM_SHARED((_NPAD, _D), jnp.float32),
            pltpu.SemaphoreType.DMA,
            pltpu.SemaphoreType.DMA,
            pltpu.SemaphoreType.DMA,
        ],
        compiler_params=pltpu.CompilerParams(use_tc_tiling_on_sc=False),
    )
    return kern(ys2, src_idx, dst_idx)


def _idx_body(s_ref, d_ref, so_ref, do_ref):
    rows = _E // _CH
    so_ref[0, :rows, :] = s_ref[0]
    so_ref[0, rows:, :] = jnp.zeros((_EP // _CH - rows, _CH), jnp.int32)
    do_ref[0, :rows, :] = d_ref[0]
    do_ref[0, rows:, :] = jnp.full((_EP // _CH - rows, _CH), _DUMMY, jnp.int32)


def _idx_prep(srcs, dsts):
    rows = _E // _CH
    return pl.pallas_call(
        _idx_body,
        grid=(2,),
        in_specs=[
            pl.BlockSpec((1, rows, _CH), lambda i: (i, 0, 0)),
            pl.BlockSpec((1, rows, _CH), lambda i: (i, 0, 0)),
        ],
        out_specs=[
            pl.BlockSpec((1, _EP // _CH, _CH), lambda i: (i, 0, 0)),
            pl.BlockSpec((1, _EP // _CH, _CH), lambda i: (i, 0, 0)),
        ],
        out_shape=[
            jax.ShapeDtypeStruct((2, _EP // _CH, _CH), jnp.int32),
            jax.ShapeDtypeStruct((2, _EP // _CH, _CH), jnp.int32),
        ],
    )(srcs, dsts)


def _lin2_body(h_ref, ab_ref, w_ref, o_ref):
    k = h_ref.shape[1] // 2
    a = ab_ref[0:1, :]
    b = ab_ref[1:2, :]
    he = h_ref[:, :k] * a + b
    ho = h_ref[:, k:] * a + b
    w = w_ref[...]
    o_ref[0] = jnp.dot(he, w, preferred_element_type=jnp.float32)
    o_ref[1] = jnp.dot(ho, w, preferred_element_type=jnp.float32)


def _lin2(h, ab, w):
    k = w.shape[0]
    return pl.pallas_call(
        _lin2_body,
        grid=(_NB,),
        in_specs=[
            pl.BlockSpec((_RP, 2 * k), lambda i: (i, 0)),
            pl.BlockSpec((2, k), lambda i: (0, 0)),
            pl.BlockSpec((k, 2 * _D), lambda i: (0, 0)),
        ],
        out_specs=pl.BlockSpec((2, _RP, 2 * _D), lambda i: (0, i, 0)),
        out_shape=jax.ShapeDtypeStruct((2, _N // 2, 2 * _D), jnp.float32),
    )(h, ab, w)


def _post_body(ys_ref, s1_ref, s2_ref, eb_ref, b1_ref, w2_ref, b2_ref,
               wm1_ref, bm1_ref, wm2_ref, bm2_ref, gb_ref,
               r_ref, ab_ref, stat_ref):
    i = pl.program_id(0)
    scat_e = jnp.concatenate([s1_ref[0][:, :_D], s2_ref[0][:, :_D]], axis=1)
    scat_o = jnp.concatenate([s1_ref[0][:, _D:], s2_ref[0][:, _D:]], axis=1)
    eb = eb_ref[...]
    b1 = b1_ref[...]
    w2 = w2_ref[...]
    b2 = b2_ref[...]
    wm1 = wm1_ref[...]
    bm1 = bm1_ref[...]
    wm2 = wm2_ref[...]
    bm2 = bm2_ref[...]

    def half(ys, scat):
        z = jnp.maximum(ys * eb + scat + b1, 0.0)
        u = jnp.maximum(
            jnp.dot(z, w2, preferred_element_type=jnp.float32) + b2, 0.0)
        v = jnp.maximum(
            jnp.dot(u, wm1, preferred_element_type=jnp.float32) + bm1, 0.0)
        return jnp.dot(v, wm2, preferred_element_type=jnp.float32) + bm2

    re = half(ys_ref[0], scat_e)
    ro = half(ys_ref[1], scat_o)
    r_ref[...] = jnp.concatenate([re, ro], axis=1)

    @pl.when(i == 0)
    def _():
        stat_ref[...] = jnp.zeros_like(stat_ref)

    stat_ref[0:1, :] += (jnp.sum(re, axis=0, keepdims=True)
                         + jnp.sum(ro, axis=0, keepdims=True))
    stat_ref[1:2, :] += (jnp.sum(re * re, axis=0, keepdims=True)
                         + jnp.sum(ro * ro, axis=0, keepdims=True))

    @pl.when(i == _NB - 1)
    def _():
        mu = stat_ref[0:1, :] * (1.0 / _N)
        var = stat_ref[1:2, :] * (1.0 / _N) - mu * mu
        a = gb_ref[0:1, :] * lax.rsqrt(var + 1e-5)
        ab_ref[0:1, :] = a
        ab_ref[1:2, :] = gb_ref[1:2, :] - mu * a


def _post(ys, agg, eb, b1r, w2bd, b2r, wm1, bm1r, wm2, bm2r, gb):
    return pl.pallas_call(
        _post_body,
        grid=(_NB,),
        in_specs=[
            pl.BlockSpec((2, _RP, 2 * _D), lambda i: (0, i, 0)),
            pl.BlockSpec((1, _RP, 2 * _D), lambda i: (0, i, 0)),
            pl.BlockSpec((1, _RP, 2 * _D), lambda i: (1, i, 0)),
            pl.BlockSpec((1, 2 * _D), lambda i: (0, 0)),
            pl.BlockSpec((1, 2 * _D), lambda i: (0, 0)),
            pl.BlockSpec((2 * _D, 2 * _D), lambda i: (0, 0)),
            pl.BlockSpec((1, 2 * _D), lambda i: (0, 0)),
            pl.BlockSpec((2 * _D, _D), lambda i: (0, 0)),
            pl.BlockSpec((1, _D), lambda i: (0, 0)),
            pl.BlockSpec((_D, _D), lambda i: (0, 0)),
            pl.BlockSpec((1, _D), lambda i: (0, 0)),
            pl.BlockSpec((2, _D), lambda i: (0, 0)),
        ],
        out_specs=[
            pl.BlockSpec((_RP, 2 * _D), lambda i: (i, 0)),
            pl.BlockSpec((2, _D), lambda i: (0, 0)),
        ],
        out_shape=[
            jax.ShapeDtypeStruct((_N // 2, 2 * _D), jnp.float32),
            jax.ShapeDtypeStruct((2, _D), jnp.float32),
        ],
        scratch_shapes=[pltpu.VMEM((2, _D), jnp.float32)],
    )(ys, agg, agg, eb, b1r, w2bd, b2r, wm1, bm1r, wm2, bm2r, gb)


def _pool_body(b2_ref, bp_ref, r1_ref, r2_ref, r3_ref, r4_ref, xa_ref,
               p_ref, pa_ref, cnt_ref):
    i = pl.program_id(0)

    @pl.when(i == 0)
    def _():
        p_ref[...] = jnp.zeros_like(p_ref)
        pa_ref[...] = jnp.zeros_like(pa_ref)
        cnt_ref[...] = jnp.zeros_like(cnt_ref)

    dn = (((0,), (0,)), ((), ()))
    iota2 = lax.broadcasted_iota(jnp.int32, (2 * _RP, _G), 1)
    iota1 = lax.broadcasted_iota(jnp.int32, (_RP, _G), 1)
    oh = (iota2 == b2_ref[:, 0:1]).astype(jnp.float32)
    oha = (iota2 == b2_ref[:, 1:2]).astype(jnp.float32)
    ohe = (iota1 == bp_ref[:, 0:1]).astype(jnp.float32)
    oho = (iota1 == bp_ref[:, 2:3]).astype(jnp.float32)
    for l, rref in enumerate((r1_ref, r2_ref, r3_ref, r4_ref)):
        rp = rref[...]
        p_ref[:, l * _D:(l + 1) * _D] += (
            lax.dot_general(ohe, rp[:, :_D], dn,
                            preferred_element_type=jnp.float32)
            + lax.dot_general(oho, rp[:, _D:], dn,
                              preferred_element_type=jnp.float32))
    pa_ref[...] += lax.dot_general(
        oha, xa_ref[...], dn, preferred_element_type=jnp.float32)
    ones8 = jnp.ones((2 * _RP, 8), jnp.float32)
    cnt_ref[:, 0:8] += lax.dot_general(
        oh, ones8, dn, preferred_element_type=jnp.float32)
    cnt_ref[:, 8:16] += lax.dot_general(
        oha, ones8, dn, preferred_element_type=jnp.float32)


def _pool(batch2, bp, r1, r2, r3, r4, xap):
    fa = xap.shape[1]
    rspec = pl.BlockSpec((_RP, 2 * _D), lambda i: (i, 0))
    return pl.pallas_call(
        _pool_body,
        grid=(_NB,),
        in_specs=[
            pl.BlockSpec((2 * _RP, 2), lambda i: (i, 0)),
            pl.BlockSpec((_RP, 4), lambda i: (i, 0)),
            rspec, rspec, rspec, rspec,
            pl.BlockSpec((2 * _RP, fa), lambda i: (i, 0)),
        ],
        out_specs=[
            pl.BlockSpec((_G, 4 * _D), lambda i: (0, 0)),
            pl.BlockSpec((_G, fa), lambda i: (0, 0)),
            pl.BlockSpec((_G, 16), lambda i: (0, 0)),
        ],
        out_shape=[
            jax.ShapeDtypeStruct((_G, 4 * _D), jnp.float32),
            jax.ShapeDtypeStruct((_G, fa), jnp.float32),
            jax.ShapeDtypeStruct((_G, 16), jnp.float32),
        ],
    )(batch2, bp, r1, r2, r3, r4, xap)


def _final_body(p_ref, pa_ref, cnt_ref, a_ref, b_ref, w1a_ref, w1b_ref,
                fb1_ref, w2_ref, fb2_ref, w3_ref, fb3_ref, w4_ref, fb4_ref,
                o_ref):
    ca = cnt_ref[:, 0:1]
    cb = cnt_ref[:, 8:9]
    mask = (ca > 0.0).astype(jnp.float32)
    avg = p_ref[...] / jnp.maximum(ca, 1.0)
    xg = (avg * a_ref[...] + b_ref[...]) * mask
    xag = pa_ref[...] / jnp.maximum(cb, 1.0)
    z = jnp.maximum(
        jnp.dot(xg, w1a_ref[...], preferred_element_type=jnp.float32)
        + jnp.dot(xag, w1b_ref[...], preferred_element_type=jnp.float32)
        + fb1_ref[...], 0.0)
    z = jnp.maximum(
        jnp.dot(z, w2_ref[...], preferred_element_type=jnp.float32)
        + fb2_ref[...], 0.0)
    z = jnp.maximum(
        jnp.dot(z, w3_ref[...], preferred_element_type=jnp.float32)
        + fb3_ref[...], 0.0)
    o_ref[...] = (jnp.dot(z, w4_ref[...], preferred_element_type=jnp.float32)
                  + fb4_ref[...])


def _final(p, pa, cnt, arow, brow, w1a, w1b, fb1, w2, fb2, w3, fb3, w4, fb4):
    fa = pa.shape[1]
    full = lambda shape: pl.BlockSpec(shape, lambda: tuple(0 for _ in shape))
    return pl.pallas_call(
        _final_body,
        grid=(),
        in_specs=[
            full((_G, 4 * _D)), full((_G, fa)), full((_G, 16)),
            full((1, 4 * _D)), full((1, 4 * _D)),
            full((4 * _D, _D)), full((fa, _D)), full((1, _D)),
            full((_D, _D)), full((1, _D)),
            full((_D, _D)), full((1, _D)),
            full((_D, 2 * _D)), full((1, 2 * _D)),
        ],
        out_specs=full((_G, 2 * _D)),
        out_shape=jax.ShapeDtypeStruct((_G, 2 * _D), jnp.float32),
    )(p, pa, cnt, arow, brow, w1a, w1b, fb1, w2, fb2, w3, fb3, w4, fb4)


def kernel(x, x_all, edge_index_1, edge_index_2, batch, batch_all, params):
    f32 = jnp.float32
    p = params

    def prep(ei, off):
        n = ei[0]
        src = n + off + (n % 2) * (_N - 1)
        return src.reshape(_E // _CH, _CH), ei[1].reshape(_E // _CH, _CH)

    s1, d1 = prep(edge_index_1, 0)
    s2, d2 = prep(edge_index_2, 1)
    src_idx, dst_idx = _idx_prep(jnp.stack([s1, s2]), jnp.stack([d1, d2]))
    src_idx = src_idx.reshape(2, 16, _NCH, _CH)
    dst_idx = dst_idx.reshape(2, 16, _NCH, _CH)

    xp = jnp.pad(x, ((0, 0), (0, 256 - _FIN))).reshape(_N // 2, 512)
    xap = jnp.pad(x_all, ((0, 0), (0, 768 - _FALL)))
    batch2 = jnp.stack([batch, batch_all], axis=1)
    bp = batch2.reshape(_N // 2, 4)

    h = xp
    ab = jnp.concatenate(
        [jnp.ones((1, 256), f32), jnp.zeros((1, 256), f32)], axis=0)
    a_parts, b_parts, rs = [], [], []
    for L in range(1, 5):
        c1 = p["conv%d_1" % L]
        c2 = p["conv%d_2" % L]
        m = p["mlp%d" % L]
        bn = p["bn%d" % L]
        w1 = jnp.concatenate([c1["W1"], c2["W1"]], axis=1)
        if L == 1:
            w1 = jnp.pad(w1, ((0, 256 - _FIN), (0, 0)))
        ysp = _lin2(h, ab, w1)
        agg = _seg_sum(ysp.reshape(2 * _N, _D), src_idx, dst_idx)
        agg = agg.reshape(2, _NPAD // 2, 2 * _D)
        eb = jnp.concatenate([jnp.full((1, _D), 1.0 + c1["eps"], f32),
                              jnp.full((1, _D), 1.0 + c2["eps"], f32)], axis=1)
        b1r = jnp.concatenate([c1["b1"], c2["b1"]]).reshape(1, 2 * _D)
        w2bd = (jnp.zeros((2 * _D, 2 * _D), f32)
                .at[:_D, :_D].set(c1["W2"]).at[_D:, _D:].set(c2["W2"]))
        b2r = jnp.concatenate([c1["b2"], c2["b2"]]).reshape(1, 2 * _D)
        gb = jnp.stack([bn["gamma"], bn["beta"]])
        r, ab = _post(ysp, agg, eb, b1r, w2bd, b2r,
                      m["W1"], m["b1"].reshape(1, _D),
                      m["W2"], m["b2"].reshape(1, _D), gb)
        a_parts.append(ab[0:1, :])
        b_parts.append(ab[1:2, :])
        rs.append(r)
        h = r

    pg, pa, cnt = _pool(batch2, bp, rs[0], rs[1], rs[2], rs[3], xap)
    arow = jnp.concatenate(a_parts, axis=1)
    brow = jnp.concatenate(b_parts, axis=1)
    w1a = p["fc1"]["W"][:4 * _D]
    w1b = jnp.pad(p["fc1"]["W"][4 * _D:], ((0, 768 - _FALL), (0, 0)))
    fb1 = p["fc1"]["b"].reshape(1, _D)
    w4 = jnp.pad(p["fc4"]["W"], ((0, 0), (0, 2 * _D - 1)))
    fb4 = jnp.pad(p["fc4"]["b"], (0, 2 * _D - 1)).reshape(1, 2 * _D)
    out = _final(pg, pa, cnt, arow, brow, w1a, w1b, fb1,
                 p["fc2"]["W"], p["fc2"]["b"].reshape(1, _D),
                 p["fc3"]["W"], p["fc3"]["b"].reshape(1, _D), w4, fb4)
    return out[:, 0]


# speedup vs baseline: 7.1913x; 1.0719x over previous
import jax
import jax.numpy as jnp
from jax import lax
from jax.experimental import pallas as pl
from jax.experimental.pallas import tpu as pltpu
from jax.experimental.pallas import tpu_sc as plsc

_N = 10000
_E = 320000
_D = 64
_G = 64
_FIN = 242
_FALL = 652

_NPAD = 10240
_DUMMY = 10200
_CH = 128
_NCH = 157
_EP = 16 * _NCH * _CH
_RP = 200
_NB = _N // (2 * _RP)


def _seg_body(ys, src_h, dst_h, agg, src_v, dst_v, rows0, rows1, acc,
              sem_i, sem0, sem1):
    c = lax.axis_index("c")
    s = lax.axis_index("s")
    cp_si = pltpu.async_copy(src_h.at[c, s], src_v, sem_i)
    cp_di = pltpu.async_copy(dst_h.at[c, s], dst_v, sem_i)

    @pl.loop(0, _CH)
    def _zero(r):
        for q in range(_D // 16):
            rows0[r, pl.ds(q * 16, 16)] = jnp.zeros((16,), jnp.float32)

    for k in range(_NPAD // 16 // _CH):
        pltpu.sync_copy(rows0, acc.at[pl.ds((s * 5 + k) * _CH, _CH)])
    cp_si.wait()
    cp_di.wait()
    plsc.subcore_barrier()

    @pl.loop(0, _NCH)
    def _edges(j):
        pltpu.async_copy(ys.at[src_v.at[j]], rows0, sem0).wait()
        pltpu.sync_copy(rows0, acc.at[dst_v.at[j]], add=True)

    plsc.subcore_barrier()
    rpt = _NPAD // 16
    pltpu.sync_copy(acc.at[pl.ds(s * rpt, rpt)],
                    agg.at[c, pl.ds(s * rpt, rpt)])


def _seg_sum(ys2, src_idx, dst_idx):
    mesh = plsc.VectorSubcoreMesh(core_axis_name="c", subcore_axis_name="s",
                                  num_cores=2, num_subcores=16)
    kern = pl.kernel(
        _seg_body,
        out_type=jax.ShapeDtypeStruct((2, _NPAD, _D), jnp.float32),
        mesh=mesh,
        scratch_types=[
            pltpu.VMEM((_NCH, _CH), jnp.int32),
            pltpu.VMEM((_NCH, _CH), jnp.int32),
            pltpu.VMEM((_CH, _D), jnp.float32),
            pltpu.VMEM((_CH, _D), jnp.float32),
            pltpu.VMEM_SHARED((_NPAD, _D), jnp.float32),
            pltpu.SemaphoreType.DMA,
            pltpu.SemaphoreType.DMA,
            pltpu.SemaphoreType.DMA,
        ],
        compiler_params=pltpu.CompilerParams(use_tc_tiling_on_sc=False),
    )
    return kern(ys2, src_idx, dst_idx)


def _idx_body(s_ref, d_ref, so_ref, do_ref):
    rows = _E // _CH
    so_ref[0, :rows, :] = s_ref[0]
    so_ref[0, rows:, :] = jnp.zeros((_EP // _CH - rows, _CH), jnp.int32)
    do_ref[0, :rows, :] = d_ref[0]
    do_ref[0, rows:, :] = jnp.full((_EP // _CH - rows, _CH), _DUMMY, jnp.int32)


def _idx_prep(srcs, dsts):
    rows = _E // _CH
    return pl.pallas_call(
        _idx_body,
        grid=(2,),
        in_specs=[
            pl.BlockSpec((1, rows, _CH), lambda i: (i, 0, 0)),
            pl.BlockSpec((1, rows, _CH), lambda i: (i, 0, 0)),
        ],
        out_specs=[
            pl.BlockSpec((1, _EP // _CH, _CH), lambda i: (i, 0, 0)),
            pl.BlockSpec((1, _EP // _CH, _CH), lambda i: (i, 0, 0)),
        ],
        out_shape=[
            jax.ShapeDtypeStruct((2, _EP // _CH, _CH), jnp.int32),
            jax.ShapeDtypeStruct((2, _EP // _CH, _CH), jnp.int32),
        ],
    )(srcs, dsts)


def _lin2_body(h_ref, ab_ref, w_ref, o_ref):
    k = h_ref.shape[1] // 2
    a = ab_ref[0:1, :]
    b = ab_ref[1:2, :]
    he = h_ref[:, :k] * a + b
    ho = h_ref[:, k:] * a + b
    w = w_ref[...]
    o_ref[0] = jnp.dot(he, w, preferred_element_type=jnp.float32)
    o_ref[1] = jnp.dot(ho, w, preferred_element_type=jnp.float32)


def _lin2(h, ab, w):
    k = w.shape[0]
    return pl.pallas_call(
        _lin2_body,
        grid=(_NB,),
        in_specs=[
            pl.BlockSpec((_RP, 2 * k), lambda i: (i, 0)),
            pl.BlockSpec((2, k), lambda i: (0, 0)),
            pl.BlockSpec((k, 2 * _D), lambda i: (0, 0)),
        ],
        out_specs=pl.BlockSpec((2, _RP, 2 * _D), lambda i: (0, i, 0)),
        out_shape=jax.ShapeDtypeStruct((2, _N // 2, 2 * _D), jnp.float32),
    )(h, ab, w)


def _post_body(ys_ref, s1_ref, s2_ref, eb_ref, b1_ref, w2_ref, b2_ref,
               wm1_ref, bm1_ref, wm2_ref, bm2_ref, gb_ref,
               r_ref, ab_ref, stat_ref):
    i = pl.program_id(0)
    scat_e = jnp.concatenate([s1_ref[0][:, :_D], s2_ref[0][:, :_D]], axis=1)
    scat_o = jnp.concatenate([s1_ref[0][:, _D:], s2_ref[0][:, _D:]], axis=1)
    eb = eb_ref[...]
    b1 = b1_ref[...]
    w2 = w2_ref[...]
    b2 = b2_ref[...]
    wm1 = wm1_ref[...]
    bm1 = bm1_ref[...]
    wm2 = wm2_ref[...]
    bm2 = bm2_ref[...]

    def half(ys, scat):
        z = jnp.maximum(ys * eb + scat + b1, 0.0)
        u = jnp.maximum(
            jnp.dot(z, w2, preferred_element_type=jnp.float32) + b2, 0.0)
        v = jnp.maximum(
            jnp.dot(u, wm1, preferred_element_type=jnp.float32) + bm1, 0.0)
        return jnp.dot(v, wm2, preferred_element_type=jnp.float32) + bm2

    re = half(ys_ref[0], scat_e)
    ro = half(ys_ref[1], scat_o)
    r_ref[...] = jnp.concatenate([re, ro], axis=1)

    @pl.when(i == 0)
    def _():
        stat_ref[...] = jnp.zeros_like(stat_ref)

    stat_ref[0:1, :] += (jnp.sum(re, axis=0, keepdims=True)
                         + jnp.sum(ro, axis=0, keepdims=True))
    stat_ref[1:2, :] += (jnp.sum(re * re, axis=0, keepdims=True)
                         + jnp.sum(ro * ro, axis=0, keepdims=True))

    @pl.when(i == _NB - 1)
    def _():
        mu = stat_ref[0:1, :] * (1.0 / _N)
        var = stat_ref[1:2, :] * (1.0 / _N) - mu * mu
        a = gb_ref[0:1, :] * lax.rsqrt(var + 1e-5)
        ab_ref[0:1, :] = a
        ab_ref[1:2, :] = gb_ref[1:2, :] - mu * a


def _post(ys, agg, eb, b1r, w2bd, b2r, wm1, bm1r, wm2, bm2r, gb):
    return pl.pallas_call(
        _post_body,
        grid=(_NB,),
        in_specs=[
            pl.BlockSpec((2, _RP, 2 * _D), lambda i: (0, i, 0)),
            pl.BlockSpec((1, _RP, 2 * _D), lambda i: (0, i, 0)),
            pl.BlockSpec((1, _RP, 2 * _D), lambda i: (1, i, 0)),
            pl.BlockSpec((1, 2 * _D), lambda i: (0, 0)),
            pl.BlockSpec((1, 2 * _D), lambda i: (0, 0)),
            pl.BlockSpec((2 * _D, 2 * _D), lambda i: (0, 0)),
            pl.BlockSpec((1, 2 * _D), lambda i: (0, 0)),
            pl.BlockSpec((2 * _D, _D), lambda i: (0, 0)),
            pl.BlockSpec((1, _D), lambda i: (0, 0)),
            pl.BlockSpec((_D, _D), lambda i: (0, 0)),
            pl.BlockSpec((1, _D), lambda i: (0, 0)),
            pl.BlockSpec((2, _D), lambda i: (0, 0)),
        ],
        out_specs=[
            pl.BlockSpec((_RP, 2 * _D), lambda i: (i, 0)),
            pl.BlockSpec((2, _D), lambda i: (0, 0)),
        ],
        out_shape=[
            jax.ShapeDtypeStruct((_N // 2, 2 * _D), jnp.float32),
            jax.ShapeDtypeStruct((2, _D), jnp.float32),
        ],
        scratch_shapes=[pltpu.VMEM((2, _D), jnp.float32)],
    )(ys, agg, agg, eb, b1r, w2bd, b2r, wm1, bm1r, wm2, bm2r, gb)


def _pool_body(b2_ref, bp_ref, r1_ref, r2_ref, r3_ref, r4_ref, xa_ref,
               p_ref, pa_ref, cnt_ref):
    i = pl.program_id(0)

    @pl.when(i == 0)
    def _():
        p_ref[...] = jnp.zeros_like(p_ref)
        pa_ref[...] = jnp.zeros_like(pa_ref)
        cnt_ref[...] = jnp.zeros_like(cnt_ref)

    dn = (((0,), (0,)), ((), ()))
    iota2 = lax.broadcasted_iota(jnp.int32, (2 * _RP, _G), 1)
    iota1 = lax.broadcasted_iota(jnp.int32, (_RP, _G), 1)
    oh = (iota2 == b2_ref[:, 0:1]).astype(jnp.float32)
    oha = (iota2 == b2_ref[:, 1:2]).astype(jnp.float32)
    ohe = (iota1 == bp_ref[:, 0:1]).astype(jnp.float32)
    oho = (iota1 == bp_ref[:, 2:3]).astype(jnp.float32)
    for l, rref in enumerate((r1_ref, r2_ref, r3_ref, r4_ref)):
        rp = rref[...]
        p_ref[:, l * _D:(l + 1) * _D] += (
            lax.dot_general(ohe, rp[:, :_D], dn,
                            preferred_element_type=jnp.float32)
            + lax.dot_general(oho, rp[:, _D:], dn,
                              preferred_element_type=jnp.float32))
    pa_ref[...] += lax.dot_general(
        oha, xa_ref[...], dn, preferred_element_type=jnp.float32)
    ones8 = jnp.ones((2 * _RP, 8), jnp.float32)
    cnt_ref[:, 0:8] += lax.dot_general(
        oh, ones8, dn, preferred_element_type=jnp.float32)
    cnt_ref[:, 8:16] += lax.dot_general(
        oha, ones8, dn, preferred_element_type=jnp.float32)


def _pool(batch2, bp, r1, r2, r3, r4, xap):
    fa = xap.shape[1]
    rspec = pl.BlockSpec((_RP, 2 * _D), lambda i: (i, 0))
    return pl.pallas_call(
        _pool_body,
        grid=(_NB,),
        in_specs=[
            pl.BlockSpec((2 * _RP, 2), lambda i: (i, 0)),
            pl.BlockSpec((_RP, 4), lambda i: (i, 0)),
            rspec, rspec, rspec, rspec,
            pl.BlockSpec((2 * _RP, fa), lambda i: (i, 0)),
        ],
        out_specs=[
            pl.BlockSpec((_G, 4 * _D), lambda i: (0, 0)),
            pl.BlockSpec((_G, fa), lambda i: (0, 0)),
            pl.BlockSpec((_G, 16), lambda i: (0, 0)),
        ],
        out_shape=[
            jax.ShapeDtypeStruct((_G, 4 * _D), jnp.float32),
            jax.ShapeDtypeStruct((_G, fa), jnp.float32),
            jax.ShapeDtypeStruct((_G, 16), jnp.float32),
        ],
    )(batch2, bp, r1, r2, r3, r4, xap)


def _final_body(p_ref, pa_ref, cnt_ref, a_ref, b_ref, w1a_ref, w1b_ref,
                fb1_ref, w2_ref, fb2_ref, w3_ref, fb3_ref, w4_ref, fb4_ref,
                o_ref):
    ca = cnt_ref[:, 0:1]
    cb = cnt_ref[:, 8:9]
    mask = (ca > 0.0).astype(jnp.float32)
    avg = p_ref[...] / jnp.maximum(ca, 1.0)
    xg = (avg * a_ref[...] + b_ref[...]) * mask
    xag = pa_ref[...] / jnp.maximum(cb, 1.0)
    z = jnp.maximum(
        jnp.dot(xg, w1a_ref[...], preferred_element_type=jnp.float32)
        + jnp.dot(xag, w1b_ref[...], preferred_element_type=jnp.float32)
        + fb1_ref[...], 0.0)
    z = jnp.maximum(
        jnp.dot(z, w2_ref[...], preferred_element_type=jnp.float32)
        + fb2_ref[...], 0.0)
    z = jnp.maximum(
        jnp.dot(z, w3_ref[...], preferred_element_type=jnp.float32)
        + fb3_ref[...], 0.0)
    o_ref[...] = (jnp.dot(z, w4_ref[...], preferred_element_type=jnp.float32)
                  + fb4_ref[...])


def _final(p, pa, cnt, arow, brow, w1a, w1b, fb1, w2, fb2, w3, fb3, w4, fb4):
    fa = pa.shape[1]
    full = lambda shape: pl.BlockSpec(shape, lambda: tuple(0 for _ in shape))
    return pl.pallas_call(
        _final_body,
        grid=(),
        in_specs=[
            full((_G, 4 * _D)), full((_G, fa)), full((_G, 16)),
            full((1, 4 * _D)), full((1, 4 * _D)),
            full((4 * _D, _D)), full((fa, _D)), full((1, _D)),
            full((_D, _D)), full((1, _D)),
            full((_D, _D)), full((1, _D)),
            full((_D, 2 * _D)), full((1, 2 * _D)),
        ],
        out_specs=full((_G, 2 * _D)),
        out_shape=jax.ShapeDtypeStruct((_G, 2 * _D), jnp.float32),
    )(p, pa, cnt, arow, brow, w1a, w1b, fb1, w2, fb2, w3, fb3, w4, fb4)


def kernel(x, x_all, edge_index_1, edge_index_2, batch, batch_all, params):
    f32 = jnp.float32
    p = params

    def prep(ei, off):
        n = ei[0]
        src = n + off + (n % 2) * (_N - 1)
        return src.reshape(_E // _CH, _CH), ei[1].reshape(_E // _CH, _CH)

    s1, d1 = prep(edge_index_1, 0)
    s2, d2 = prep(edge_index_2, 1)
    src_idx, dst_idx = _idx_prep(jnp.stack([s1, s2]), jnp.stack([d1, d2]))
    src_idx = src_idx.reshape(2, 16, _NCH, _CH)
    dst_idx = dst_idx.reshape(2, 16, _NCH, _CH)

    xp = jnp.pad(x, ((0, 0), (0, 256 - _FIN))).reshape(_N // 2, 512)
    xap = x_all
    batch2 = jnp.stack([batch, batch_all], axis=1)
    bp = batch2.reshape(_N // 2, 4)

    h = xp
    ab = jnp.concatenate(
        [jnp.ones((1, 256), f32), jnp.zeros((1, 256), f32)], axis=0)
    a_parts, b_parts, rs = [], [], []
    for L in range(1, 5):
        c1 = p["conv%d_1" % L]
        c2 = p["conv%d_2" % L]
        m = p["mlp%d" % L]
        bn = p["bn%d" % L]
        w1 = jnp.concatenate([c1["W1"], c2["W1"]], axis=1)
        if L == 1:
            w1 = jnp.pad(w1, ((0, 256 - _FIN), (0, 0)))
        ysp = _lin2(h, ab, w1)
        agg = _seg_sum(ysp.reshape(2 * _N, _D), src_idx, dst_idx)
        agg = agg.reshape(2, _NPAD // 2, 2 * _D)
        eb = jnp.concatenate([jnp.full((1, _D), 1.0 + c1["eps"], f32),
                              jnp.full((1, _D), 1.0 + c2["eps"], f32)], axis=1)
        b1r = jnp.concatenate([c1["b1"], c2["b1"]]).reshape(1, 2 * _D)
        w2bd = (jnp.zeros((2 * _D, 2 * _D), f32)
                .at[:_D, :_D].set(c1["W2"]).at[_D:, _D:].set(c2["W2"]))
        b2r = jnp.concatenate([c1["b2"], c2["b2"]]).reshape(1, 2 * _D)
        gb = jnp.stack([bn["gamma"], bn["beta"]])
        r, ab = _post(ysp, agg, eb, b1r, w2bd, b2r,
                      m["W1"], m["b1"].reshape(1, _D),
                      m["W2"], m["b2"].reshape(1, _D), gb)
        a_parts.append(ab[0:1, :])
        b_parts.append(ab[1:2, :])
        rs.append(r)
        h = r

    pg, pa, cnt = _pool(batch2, bp, rs[0], rs[1], rs[2], rs[3], xap)
    arow = jnp.concatenate(a_parts, axis=1)
    brow = jnp.concatenate(b_parts, axis=1)
    w1a = p["fc1"]["W"][:4 * _D]
    w1b = p["fc1"]["W"][4 * _D:]
    fb1 = p["fc1"]["b"].reshape(1, _D)
    w4 = jnp.pad(p["fc4"]["W"], ((0, 0), (0, 2 * _D - 1)))
    fb4 = jnp.pad(p["fc4"]["b"], (0, 2 * _D - 1)).reshape(1, 2 * _D)
    out = _final(pg, pa, cnt, arow, brow, w1a, w1b, fb1,
                 p["fc2"]["W"], p["fc2"]["b"].reshape(1, _D),
                 p["fc3"]["W"], p["fc3"]["b"].reshape(1, _D), w4, fb4)
    return out[:, 0]


# speedup vs baseline: 7.2047x; 1.0019x over previous
import jax
import jax.numpy as jnp
from jax import lax
from jax.experimental import pallas as pl
from jax.experimental.pallas import tpu as pltpu
from jax.experimental.pallas import tpu_sc as plsc

_N = 10000
_E = 320000
_D = 64
_G = 64
_FIN = 242
_FALL = 652

_NPAD = 10240
_DUMMY = 10200
_CH = 128
_NCH = 157
_EP = 16 * _NCH * _CH
_RP = 200
_NB = _N // (2 * _RP)


def _seg_body(ys, src_h, dst_h, agg, src_v, dst_v, rows0, rows1, acc,
              sem_i, sem0, sem1):
    c = lax.axis_index("c")
    s = lax.axis_index("s")
    cp_si = pltpu.async_copy(src_h.at[c, s], src_v, sem_i)
    cp_di = pltpu.async_copy(dst_h.at[c, s], dst_v, sem_i)

    @pl.loop(0, _CH)
    def _zero(r):
        for q in range(_D // 16):
            rows0[r, pl.ds(q * 16, 16)] = jnp.zeros((16,), jnp.float32)

    for k in range(_NPAD // 16 // _CH):
        pltpu.sync_copy(rows0, acc.at[pl.ds((s * 5 + k) * _CH, _CH)])
    cp_si.wait()
    cp_di.wait()
    plsc.subcore_barrier()

    @pl.loop(0, _NCH)
    def _edges(j):
        pltpu.async_copy(ys.at[src_v.at[j]], rows0, sem0).wait()
        pltpu.sync_copy(rows0, acc.at[dst_v.at[j]], add=True)

    plsc.subcore_barrier()
    rpt = _NPAD // 16
    pltpu.sync_copy(acc.at[pl.ds(s * rpt, rpt)],
                    agg.at[c, pl.ds(s * rpt, rpt)])


def _seg_sum(ys2, src_idx, dst_idx):
    mesh = plsc.VectorSubcoreMesh(core_axis_name="c", subcore_axis_name="s",
                                  num_cores=2, num_subcores=16)
    kern = pl.kernel(
        _seg_body,
        out_type=jax.ShapeDtypeStruct((2, _NPAD, _D), jnp.float32),
        mesh=mesh,
        scratch_types=[
            pltpu.VMEM((_NCH, _CH), jnp.int32),
            pltpu.VMEM((_NCH, _CH), jnp.int32),
            pltpu.VMEM((_CH, _D), jnp.float32),
            pltpu.VMEM((_CH, _D), jnp.float32),
            pltpu.VMEM_SHARED((_NPAD, _D), jnp.float32),
            pltpu.SemaphoreType.DMA,
            pltpu.SemaphoreType.DMA,
            pltpu.SemaphoreType.DMA,
        ],
        compiler_params=pltpu.CompilerParams(use_tc_tiling_on_sc=False),
    )
    return kern(ys2, src_idx, dst_idx)


def _idx_body(s_ref, d_ref, so_ref, do_ref):
    rows = _E // _CH
    so_ref[0, :rows, :] = s_ref[0]
    so_ref[0, rows:, :] = jnp.zeros((_EP // _CH - rows, _CH), jnp.int32)
    do_ref[0, :rows, :] = d_ref[0]
    do_ref[0, rows:, :] = jnp.full((_EP // _CH - rows, _CH), _DUMMY, jnp.int32)


def _idx_prep(srcs, dsts):
    rows = _E // _CH
    return pl.pallas_call(
        _idx_body,
        grid=(2,),
        in_specs=[
            pl.BlockSpec((1, rows, _CH), lambda i: (i, 0, 0)),
            pl.BlockSpec((1, rows, _CH), lambda i: (i, 0, 0)),
        ],
        out_specs=[
            pl.BlockSpec((1, _EP // _CH, _CH), lambda i: (i, 0, 0)),
            pl.BlockSpec((1, _EP // _CH, _CH), lambda i: (i, 0, 0)),
        ],
        out_shape=[
            jax.ShapeDtypeStruct((2, _EP // _CH, _CH), jnp.int32),
            jax.ShapeDtypeStruct((2, _EP // _CH, _CH), jnp.int32),
        ],
    )(srcs, dsts)


def _lin2_body(h_ref, ab_ref, w_ref, o_ref):
    k = h_ref.shape[1] // 2
    a = ab_ref[0:1, :]
    b = ab_ref[1:2, :]
    he = h_ref[:, :k] * a + b
    ho = h_ref[:, k:] * a + b
    w = w_ref[...]
    o_ref[0] = jnp.dot(he, w, preferred_element_type=jnp.float32)
    o_ref[1] = jnp.dot(ho, w, preferred_element_type=jnp.float32)


def _lin2(h, ab, w):
    k = w.shape[0]
    return pl.pallas_call(
        _lin2_body,
        grid=(_NB,),
        in_specs=[
            pl.BlockSpec((_RP, 2 * k), lambda i: (i, 0)),
            pl.BlockSpec((2, k), lambda i: (0, 0)),
            pl.BlockSpec((k, 2 * _D), lambda i: (0, 0)),
        ],
        out_specs=pl.BlockSpec((2, _RP, 2 * _D), lambda i: (0, i, 0)),
        out_shape=jax.ShapeDtypeStruct((2, _N // 2, 2 * _D), jnp.float32),
    )(h, ab, w)


def _post_body(ys_ref, s1_ref, s2_ref, eb_ref, b1_ref, w2_ref, b2_ref,
               wm1_ref, bm1_ref, wm2_ref, bm2_ref, gb_ref,
               r_ref, ab_ref, stat_ref):
    i = pl.program_id(0)
    scat_e = jnp.concatenate([s1_ref[0][:, :_D], s2_ref[0][:, :_D]], axis=1)
    scat_o = jnp.concatenate([s1_ref[0][:, _D:], s2_ref[0][:, _D:]], axis=1)
    eb = eb_ref[...]
    b1 = b1_ref[...]
    w2 = w2_ref[...]
    b2 = b2_ref[...]
    wm1 = wm1_ref[...]
    bm1 = bm1_ref[...]
    wm2 = wm2_ref[...]
    bm2 = bm2_ref[...]

    def half(ys, scat):
        z = jnp.maximum(ys * eb + scat + b1, 0.0)
        u = jnp.maximum(
            jnp.dot(z, w2, preferred_element_type=jnp.float32) + b2, 0.0)
        v = jnp.maximum(
            jnp.dot(u, wm1, preferred_element_type=jnp.float32) + bm1, 0.0)
        return jnp.dot(v, wm2, preferred_element_type=jnp.float32) + bm2

    re = half(ys_ref[0], scat_e)
    ro = half(ys_ref[1], scat_o)
    r_ref[...] = jnp.concatenate([re, ro], axis=1)

    @pl.when(i == 0)
    def _():
        stat_ref[...] = jnp.zeros_like(stat_ref)

    stat_ref[0:1, :] += (jnp.sum(re, axis=0, keepdims=True)
                         + jnp.sum(ro, axis=0, keepdims=True))
    stat_ref[1:2, :] += (jnp.sum(re * re, axis=0, keepdims=True)
                         + jnp.sum(ro * ro, axis=0, keepdims=True))

    @pl.when(i == _NB - 1)
    def _():
        mu = stat_ref[0:1, :] * (1.0 / _N)
        var = stat_ref[1:2, :] * (1.0 / _N) - mu * mu
        a = gb_ref[0:1, :] * lax.rsqrt(var + 1e-5)
        ab_ref[0:1, :] = a
        ab_ref[1:2, :] = gb_ref[1:2, :] - mu * a


def _post(ys, agg, eb, b1r, w2bd, b2r, wm1, bm1r, wm2, bm2r, gb):
    return pl.pallas_call(
        _post_body,
        grid=(_NB,),
        in_specs=[
            pl.BlockSpec((2, _RP, 2 * _D), lambda i: (0, i, 0)),
            pl.BlockSpec((1, _RP, 2 * _D), lambda i: (0, i, 0)),
            pl.BlockSpec((1, _RP, 2 * _D), lambda i: (1, i, 0)),
            pl.BlockSpec((1, 2 * _D), lambda i: (0, 0)),
            pl.BlockSpec((1, 2 * _D), lambda i: (0, 0)),
            pl.BlockSpec((2 * _D, 2 * _D), lambda i: (0, 0)),
            pl.BlockSpec((1, 2 * _D), lambda i: (0, 0)),
            pl.BlockSpec((2 * _D, _D), lambda i: (0, 0)),
            pl.BlockSpec((1, _D), lambda i: (0, 0)),
            pl.BlockSpec((_D, _D), lambda i: (0, 0)),
            pl.BlockSpec((1, _D), lambda i: (0, 0)),
            pl.BlockSpec((2, _D), lambda i: (0, 0)),
        ],
        out_specs=[
            pl.BlockSpec((_RP, 2 * _D), lambda i: (i, 0)),
            pl.BlockSpec((2, _D), lambda i: (0, 0)),
        ],
        out_shape=[
            jax.ShapeDtypeStruct((_N // 2, 2 * _D), jnp.float32),
            jax.ShapeDtypeStruct((2, _D), jnp.float32),
        ],
        scratch_shapes=[pltpu.VMEM((2, _D), jnp.float32)],
    )(ys, agg, agg, eb, b1r, w2bd, b2r, wm1, bm1r, wm2, bm2r, gb)


def _pool_body(b2_ref, bp_ref, r1_ref, r2_ref, r3_ref, r4_ref,
               p_ref, cnt_ref):
    i = pl.program_id(0)

    @pl.when(i == 0)
    def _():
        p_ref[...] = jnp.zeros_like(p_ref)
        cnt_ref[...] = jnp.zeros_like(cnt_ref)

    dn = (((0,), (0,)), ((), ()))
    iota2 = lax.broadcasted_iota(jnp.int32, (2 * _RP, _G), 1)
    iota1 = lax.broadcasted_iota(jnp.int32, (_RP, _G), 1)
    oh = (iota2 == b2_ref[:, 0:1]).astype(jnp.float32)
    ohe = (iota1 == bp_ref[:, 0:1]).astype(jnp.float32)
    oho = (iota1 == bp_ref[:, 2:3]).astype(jnp.float32)
    for l, rref in enumerate((r1_ref, r2_ref, r3_ref, r4_ref)):
        rp = rref[...]
        p_ref[:, l * _D:(l + 1) * _D] += (
            lax.dot_general(ohe, rp[:, :_D], dn,
                            preferred_element_type=jnp.float32)
            + lax.dot_general(oho, rp[:, _D:], dn,
                              preferred_element_type=jnp.float32))
    ones8 = jnp.ones((2 * _RP, 8), jnp.float32)
    cnt_ref[...] += lax.dot_general(
        oh, ones8, dn, preferred_element_type=jnp.float32)


def _pool(batch2, bp, r1, r2, r3, r4):
    rspec = pl.BlockSpec((_RP, 2 * _D), lambda i: (i, 0))
    return pl.pallas_call(
        _pool_body,
        grid=(_NB,),
        in_specs=[
            pl.BlockSpec((2 * _RP, 2), lambda i: (i, 0)),
            pl.BlockSpec((_RP, 4), lambda i: (i, 0)),
            rspec, rspec, rspec, rspec,
        ],
        out_specs=[
            pl.BlockSpec((_G, 4 * _D), lambda i: (0, 0)),
            pl.BlockSpec((_G, 8), lambda i: (0, 0)),
        ],
        out_shape=[
            jax.ShapeDtypeStruct((_G, 4 * _D), jnp.float32),
            jax.ShapeDtypeStruct((_G, 8), jnp.float32),
        ],
    )(batch2, bp, r1, r2, r3, r4)


def _poolx_body(b2_ref, xa_ref, pa_ref, cnt_ref):
    i = pl.program_id(0)

    @pl.when(i == 0)
    def _():
        pa_ref[...] = jnp.zeros_like(pa_ref)
        cnt_ref[...] = jnp.zeros_like(cnt_ref)

    dn = (((0,), (0,)), ((), ()))
    iota2 = lax.broadcasted_iota(jnp.int32, (2 * _RP, _G), 1)
    oha = (iota2 == b2_ref[:, 1:2]).astype(jnp.float32)
    pa_ref[...] += lax.dot_general(
        oha, xa_ref[...], dn, preferred_element_type=jnp.float32)
    ones8 = jnp.ones((2 * _RP, 8), jnp.float32)
    cnt_ref[...] += lax.dot_general(
        oha, ones8, dn, preferred_element_type=jnp.float32)


def _poolx(batch2, xap):
    fa = xap.shape[1]
    return pl.pallas_call(
        _poolx_body,
        grid=(_NB,),
        in_specs=[
            pl.BlockSpec((2 * _RP, 2), lambda i: (i, 0)),
            pl.BlockSpec((2 * _RP, fa), lambda i: (i, 0)),
        ],
        out_specs=[
            pl.BlockSpec((_G, fa), lambda i: (0, 0)),
            pl.BlockSpec((_G, 8), lambda i: (0, 0)),
        ],
        out_shape=[
            jax.ShapeDtypeStruct((_G, fa), jnp.float32),
            jax.ShapeDtypeStruct((_G, 8), jnp.float32),
        ],
    )(batch2, xap)


def _final_body(p_ref, pa_ref, cb_ref, ca_ref, a_ref, b_ref, w1a_ref,
                w1b_ref, fb1_ref, w2_ref, fb2_ref, w3_ref, fb3_ref, w4_ref,
                fb4_ref, o_ref):
    ca = ca_ref[:, 0:1]
    cb = cb_ref[:, 0:1]
    mask = (ca > 0.0).astype(jnp.float32)
    avg = p_ref[...] / jnp.maximum(ca, 1.0)
    xg = (avg * a_ref[...] + b_ref[...]) * mask
    xag = pa_ref[...] / jnp.maximum(cb, 1.0)
    z = jnp.maximum(
        jnp.dot(xg, w1a_ref[...], preferred_element_type=jnp.float32)
        + jnp.dot(xag, w1b_ref[...], preferred_element_type=jnp.float32)
        + fb1_ref[...], 0.0)
    z = jnp.maximum(
        jnp.dot(z, w2_ref[...], preferred_element_type=jnp.float32)
        + fb2_ref[...], 0.0)
    z = jnp.maximum(
        jnp.dot(z, w3_ref[...], preferred_element_type=jnp.float32)
        + fb3_ref[...], 0.0)
    o_ref[...] = (jnp.dot(z, w4_ref[...], preferred_element_type=jnp.float32)
                  + fb4_ref[...])


def _final(p, pa, cb, ca, arow, brow, w1a, w1b, fb1, w2, fb2, w3, fb3, w4,
           fb4):
    fa = pa.shape[1]
    full = lambda shape: pl.BlockSpec(shape, lambda: tuple(0 for _ in shape))
    return pl.pallas_call(
        _final_body,
        grid=(),
        in_specs=[
            full((_G, 4 * _D)), full((_G, fa)), full((_G, 8)), full((_G, 8)),
            full((1, 4 * _D)), full((1, 4 * _D)),
            full((4 * _D, _D)), full((fa, _D)), full((1, _D)),
            full((_D, _D)), full((1, _D)),
            full((_D, _D)), full((1, _D)),
            full((_D, 2 * _D)), full((1, 2 * _D)),
        ],
        out_specs=full((_G, 2 * _D)),
        out_shape=jax.ShapeDtypeStruct((_G, 2 * _D), jnp.float32),
    )(p, pa, cb, ca, arow, brow, w1a, w1b, fb1, w2, fb2, w3, fb3, w4, fb4)


def kernel(x, x_all, edge_index_1, edge_index_2, batch, batch_all, params):
    f32 = jnp.float32
    p = params

    def prep(ei, off):
        n = ei[0]
        src = n + off + (n % 2) * (_N - 1)
        return src.reshape(_E // _CH, _CH), ei[1].reshape(_E // _CH, _CH)

    s1, d1 = prep(edge_index_1, 0)
    s2, d2 = prep(edge_index_2, 1)
    src_idx, dst_idx = _idx_prep(jnp.stack([s1, s2]), jnp.stack([d1, d2]))
    src_idx = src_idx.reshape(2, 16, _NCH, _CH)
    dst_idx = dst_idx.reshape(2, 16, _NCH, _CH)

    xp = jnp.pad(x, ((0, 0), (0, 256 - _FIN))).reshape(_N // 2, 512)
    xap = x_all
    batch2 = jnp.stack([batch, batch_all], axis=1)
    bp = batch2.reshape(_N // 2, 4)

    pa, cnta = _poolx(batch2, xap)

    h = xp
    ab = jnp.concatenate(
        [jnp.ones((1, 256), f32), jnp.zeros((1, 256), f32)], axis=0)
    a_parts, b_parts, rs = [], [], []
    for L in range(1, 5):
        c1 = p["conv%d_1" % L]
        c2 = p["conv%d_2" % L]
        m = p["mlp%d" % L]
        bn = p["bn%d" % L]
        w1 = jnp.concatenate([c1["W1"], c2["W1"]], axis=1)
        if L == 1:
            w1 = jnp.pad(w1, ((0, 256 - _FIN), (0, 0)))
        ysp = _lin2(h, ab, w1)
        agg = _seg_sum(ysp.reshape(2 * _N, _D), src_idx, dst_idx)
        agg = agg.reshape(2, _NPAD // 2, 2 * _D)
        eb = jnp.concatenate([jnp.full((1, _D), 1.0 + c1["eps"], f32),
                              jnp.full((1, _D), 1.0 + c2["eps"], f32)], axis=1)
        b1r = jnp.concatenate([c1["b1"], c2["b1"]]).reshape(1, 2 * _D)
        w2bd = (jnp.zeros((2 * _D, 2 * _D), f32)
                .at[:_D, :_D].set(c1["W2"]).at[_D:, _D:].set(c2["W2"]))
        b2r = jnp.concatenate([c1["b2"], c2["b2"]]).reshape(1, 2 * _D)
        gb = jnp.stack([bn["gamma"], bn["beta"]])
        r, ab = _post(ysp, agg, eb, b1r, w2bd, b2r,
                      m["W1"], m["b1"].reshape(1, _D),
                      m["W2"], m["b2"].reshape(1, _D), gb)
        a_parts.append(ab[0:1, :])
        b_parts.append(ab[1:2, :])
        rs.append(r)
        h = r

    pg, cntb = _pool(batch2, bp, rs[0], rs[1], rs[2], rs[3])
    arow = jnp.concatenate(a_parts, axis=1)
    brow = jnp.concatenate(b_parts, axis=1)
    w1a = p["fc1"]["W"][:4 * _D]
    w1b = p["fc1"]["W"][4 * _D:]
    fb1 = p["fc1"]["b"].reshape(1, _D)
    w4 = jnp.pad(p["fc4"]["W"], ((0, 0), (0, 2 * _D - 1)))
    fb4 = jnp.pad(p["fc4"]["b"], (0, 2 * _D - 1)).reshape(1, 2 * _D)
    out = _final(pg, pa, cnta, cntb, arow, brow, w1a, w1b, fb1,
                 p["fc2"]["W"], p["fc2"]["b"].reshape(1, _D),
                 p["fc3"]["W"], p["fc3"]["b"].reshape(1, _D), w4, fb4)
    return out[:, 0]
